# baseline (speedup 1.0000x reference)
import functools

import jax
import jax.numpy as jnp
from jax import lax
from jax.experimental import pallas as pl
from jax.experimental.pallas import tpu as pltpu
from jax.experimental.pallas import tpu_sc as plsc

D_MODEL = 64
N_IDX = 16384 * 50

_info = plsc.get_sparse_core_info()
NC = _info.num_cores
NS = _info.num_subcores
NW = NC * NS
PER_W = N_IDX // NW
CHUNK = 640
N_CHUNKS = PER_W // CHUNK
NBUF = 2
N_OUTER = N_CHUNKS // NBUF

_mesh = plsc.VectorSubcoreMesh(core_axis_name="c", subcore_axis_name="s")


@functools.partial(
    pl.kernel,
    mesh=_mesh,
    out_type=jax.ShapeDtypeStruct((N_IDX, D_MODEL), jnp.float32),
    scratch_types=(
        [pltpu.VMEM((PER_W,), jnp.int32)]
        + [pltpu.VMEM((CHUNK, D_MODEL), jnp.float32) for _ in range(NBUF)]
        + [pltpu.SemaphoreType.DMA for _ in range(2 * NBUF)]
    ),
    compiler_params=pltpu.CompilerParams(use_tc_tiling_on_sc=False),
)
def _embed_gather(x_hbm, table_hbm, out_hbm, idx_v, *bufs):
    rows = bufs[:NBUF]
    gsem = bufs[NBUF:2 * NBUF]
    ssem = bufs[2 * NBUF:]
    wid = lax.axis_index("s") * NC + lax.axis_index("c")
    base = wid * PER_W

    pltpu.sync_copy(x_hbm.at[pl.ds(base, PER_W)], idx_v)

    def idx_at(g):
        return idx_v.at[pl.ds(g * CHUNK, CHUNK)]

    def out_at(g):
        return out_hbm.at[pl.ds(base + g * CHUNK, CHUNK)]

    for b in range(NBUF):
        pltpu.async_copy(table_hbm.at[idx_at(b)], rows[b], gsem[b])

    def outer(i, carry):
        g0 = i * NBUF
        for b in range(NBUF):
            g = g0 + b
            pltpu.make_async_copy(table_hbm.at[idx_at(g)], rows[b],
                                  gsem[b]).wait()
            pltpu.async_copy(rows[b], out_at(g), ssem[b])
            pltpu.make_async_copy(rows[b], out_at(g), ssem[b]).wait()
            pltpu.async_copy(table_hbm.at[idx_at(g + NBUF)], rows[b],
                             gsem[b])
        return carry

    lax.fori_loop(0, N_OUTER - 1, outer, 0)

    g_last = (N_OUTER - 1) * NBUF
    for b in range(NBUF):
        g = g_last + b
        pltpu.make_async_copy(table_hbm.at[idx_at(g)], rows[b],
                              gsem[b]).wait()
        pltpu.async_copy(rows[b], out_at(g), ssem[b])
    for b in range(NBUF):
        g = g_last + b
        pltpu.make_async_copy(rows[b], out_at(g), ssem[b]).wait()


def kernel(x, table):
    x_flat = x.reshape(-1).astype(jnp.int32)
    out = _embed_gather(x_flat, table)
    return out.reshape(x.shape + (table.shape[1],))


# speedup vs baseline: 1.0020x; 1.0020x over previous
import functools

import jax
import jax.numpy as jnp
from jax import lax
from jax.experimental import pallas as pl
from jax.experimental.pallas import tpu as pltpu
from jax.experimental.pallas import tpu_sc as plsc

D_MODEL = 64
N_IDX = 16384 * 50

_info = plsc.get_sparse_core_info()
NC = _info.num_cores
NS = _info.num_subcores
NW = NC * NS
PER_W = N_IDX // NW
CHUNK = 400
N_CHUNKS = PER_W // CHUNK
NBUF = 2
N_OUTER = N_CHUNKS // NBUF

_mesh = plsc.VectorSubcoreMesh(core_axis_name="c", subcore_axis_name="s")


@functools.partial(
    pl.kernel,
    mesh=_mesh,
    out_type=jax.ShapeDtypeStruct((N_IDX, D_MODEL), jnp.float32),
    scratch_types=(
        [pltpu.VMEM((PER_W,), jnp.int32),
         pltpu.VMEM_SHARED((NS, NBUF, CHUNK, D_MODEL), jnp.float32)]
        + [pltpu.VMEM((CHUNK, D_MODEL), jnp.float32) for _ in range(NBUF)]
        + [pltpu.SemaphoreType.DMA for _ in range(3 * NBUF)]
    ),
    compiler_params=pltpu.CompilerParams(use_tc_tiling_on_sc=False),
)
def _embed_gather(x_hbm, table_hbm, out_hbm, idx_v, shared, *bufs):
    rows = bufs[:NBUF]
    gsem = bufs[NBUF:2 * NBUF]
    csem = bufs[2 * NBUF:3 * NBUF]
    ssem = bufs[3 * NBUF:]
    sid = lax.axis_index("s")
    wid = sid * NC + lax.axis_index("c")
    base = wid * PER_W

    pltpu.sync_copy(x_hbm.at[pl.ds(base, PER_W)], idx_v)

    def idx_at(g):
        return idx_v.at[pl.ds(g * CHUNK, CHUNK)]

    def out_at(g):
        return out_hbm.at[pl.ds(base + g * CHUNK, CHUNK)]

    def shr(b):
        return shared.at[sid, b]

    def stage(g, b, wait_out, next_gather):
        pltpu.make_async_copy(table_hbm.at[idx_at(g)], rows[b],
                              gsem[b]).wait()
        if wait_out:
            pltpu.make_async_copy(shr(b), out_at(g - NBUF), ssem[b]).wait()
        pltpu.async_copy(rows[b], shr(b), csem[b])
        pltpu.make_async_copy(rows[b], shr(b), csem[b]).wait()
        pltpu.async_copy(shr(b), out_at(g), ssem[b])
        if next_gather:
            pltpu.async_copy(table_hbm.at[idx_at(g + NBUF)], rows[b],
                             gsem[b])

    for b in range(NBUF):
        pltpu.async_copy(table_hbm.at[idx_at(b)], rows[b], gsem[b])

    for b in range(NBUF):
        stage(b, b, wait_out=False, next_gather=True)

    def outer(i, carry):
        for b in range(NBUF):
            stage(i * NBUF + b, b, wait_out=True, next_gather=True)
        return carry

    lax.fori_loop(1, N_OUTER - 1, outer, 0)

    g_last = (N_OUTER - 1) * NBUF
    for b in range(NBUF):
        stage(g_last + b, b, wait_out=True, next_gather=False)
    for b in range(NBUF):
        pltpu.make_async_copy(shr(b), out_at(g_last + b), ssem[b]).wait()


def kernel(x, table):
    x_flat = x.reshape(-1).astype(jnp.int32)
    out = _embed_gather(x_flat, table)
    return out.reshape(x.shape + (table.shape[1],))


# speedup vs baseline: 1.0025x; 1.0005x over previous
import functools

import jax
import jax.numpy as jnp
from jax import lax
from jax.experimental import pallas as pl
from jax.experimental.pallas import tpu as pltpu
from jax.experimental.pallas import tpu_sc as plsc

D_MODEL = 64
N_IDX = 16384 * 50

_info = plsc.get_sparse_core_info()
NC = _info.num_cores
NS = _info.num_subcores
NW = NC * NS
PER_W = N_IDX // NW
CHUNK = 400
N_CHUNKS = PER_W // CHUNK
NBUF = 2
N_OUTER = N_CHUNKS // NBUF

_mesh = plsc.VectorSubcoreMesh(core_axis_name="c", subcore_axis_name="s")


@functools.partial(
    pl.kernel,
    mesh=_mesh,
    out_type=jax.ShapeDtypeStruct((N_IDX, D_MODEL), jnp.float32),
    scratch_types=(
        [pltpu.VMEM((PER_W,), jnp.int32),
         pltpu.VMEM_SHARED((NS, NBUF, CHUNK, D_MODEL), jnp.float32)]
        + [pltpu.VMEM((CHUNK, D_MODEL), jnp.float32) for _ in range(NBUF)]
        + [pltpu.SemaphoreType.DMA for _ in range(3 * NBUF + 1)]
    ),
    compiler_params=pltpu.CompilerParams(use_tc_tiling_on_sc=False),
)
def _embed_gather(x_hbm, table_hbm, out_hbm, idx_v, shared, *bufs):
    rows = bufs[:NBUF]
    gsem = bufs[NBUF:2 * NBUF]
    csem = bufs[2 * NBUF:3 * NBUF]
    ssem = bufs[3 * NBUF:4 * NBUF]
    isem = bufs[4 * NBUF]
    sid = lax.axis_index("s")
    wid = sid * NC + lax.axis_index("c")
    base = wid * PER_W

    head = NBUF * CHUNK
    pltpu.sync_copy(x_hbm.at[pl.ds(base, head)], idx_v.at[pl.ds(0, head)])
    pltpu.async_copy(x_hbm.at[pl.ds(base + head, PER_W - head)],
                     idx_v.at[pl.ds(head, PER_W - head)], isem)

    def idx_at(g):
        return idx_v.at[pl.ds(g * CHUNK, CHUNK)]

    def out_at(g):
        return out_hbm.at[pl.ds(base + g * CHUNK, CHUNK)]

    def shr(b):
        return shared.at[sid, b]

    def stage(g, b, wait_out, next_gather):
        pltpu.make_async_copy(table_hbm.at[idx_at(g)], rows[b],
                              gsem[b]).wait()
        if wait_out:
            pltpu.make_async_copy(shr(b), out_at(g - NBUF), ssem[b]).wait()
        pltpu.async_copy(rows[b], shr(b), csem[b])
        pltpu.make_async_copy(rows[b], shr(b), csem[b]).wait()
        pltpu.async_copy(shr(b), out_at(g), ssem[b])
        if next_gather:
            pltpu.async_copy(table_hbm.at[idx_at(g + NBUF)], rows[b],
                             gsem[b])

    for b in range(NBUF):
        pltpu.async_copy(table_hbm.at[idx_at(b)], rows[b], gsem[b])
    pltpu.make_async_copy(x_hbm.at[pl.ds(base + head, PER_W - head)],
                          idx_v.at[pl.ds(head, PER_W - head)], isem).wait()

    for b in range(NBUF):
        stage(b, b, wait_out=False, next_gather=True)

    def outer(i, carry):
        for b in range(NBUF):
            stage(i * NBUF + b, b, wait_out=True, next_gather=True)
        return carry

    lax.fori_loop(1, N_OUTER - 1, outer, 0)

    g_last = (N_OUTER - 1) * NBUF
    for b in range(NBUF):
        stage(g_last + b, b, wait_out=True, next_gather=False)
    for b in range(NBUF):
        pltpu.make_async_copy(shr(b), out_at(g_last + b), ssem[b]).wait()


def kernel(x, table):
    x_flat = x.reshape(-1).astype(jnp.int32)
    out = _embed_gather(x_flat, table)
    return out.reshape(x.shape + (table.shape[1],))
